# Initial kernel scaffold; baseline (speedup 1.0000x reference)
import jax
import jax.numpy as jnp
from jax.experimental import pallas as pl


def kernel(cls_token, coarse_prototypes, fine_prototypes, Wq, bq, Wp, bp, ln_g, ln_b):
    raise NotImplementedError("write your pallas kernel here")


# speedup vs baseline: 19.4301x; 19.4301x over previous
import functools

import jax
import jax.numpy as jnp
from jax import lax
from jax.experimental import pallas as pl

B = 16384
D = 128
K = 64
M = 8
KM = K * M
BLK = 512


def _prep_kernel(cp_ref, ff_ref, wp_ref, bp_ref, cn_ref, fn_ref, fp_ref):
    cp = cp_ref[:]
    cn_ref[:] = cp / jnp.maximum(
        jnp.sqrt(jnp.sum(cp * cp, axis=1, keepdims=True)), 1e-12)
    ff = ff_ref[:]
    fn_ref[:] = ff / jnp.maximum(
        jnp.sqrt(jnp.sum(ff * ff, axis=1, keepdims=True)), 1e-12)
    fp_ref[:] = lax.dot_general(
        ff, wp_ref[:], (((1,), (1,)), ((), ())),
        preferred_element_type=jnp.float32) + bp_ref[:]


def _main_kernel(cls_ref, wq_ref, bq_ref, g_ref, b_ref,
                 cn_ref, fn_ref, fp_ref,
                 prompt_ref, cw_ref, fw_ref):
    x = cls_ref[:]
    q = lax.dot_general(x, wq_ref[:], (((1,), (1,)), ((), ())),
                        preferred_element_type=jnp.float32) + bq_ref[:]
    mu = jnp.mean(q, axis=1, keepdims=True)
    c = q - mu
    var = jnp.mean(c * c, axis=1, keepdims=True)
    qn = c / jnp.sqrt(var + 1e-5) * g_ref[:] + b_ref[:]
    nrm = jnp.maximum(jnp.sqrt(jnp.sum(qn * qn, axis=1, keepdims=True)), 1e-12)
    q1 = qn / nrm

    cs = lax.dot_general(q1, cn_ref[:], (((1,), (1,)), ((), ())),
                         preferred_element_type=jnp.float32)
    ec = jnp.exp(2.0 * cs)
    cw = ec / jnp.sum(ec, axis=1, keepdims=True)

    wj = cw[:, :, None]
    wi = cw[:, None, :]
    jlt = (lax.broadcasted_iota(jnp.int32, (1, K, K), 1) <
           lax.broadcasted_iota(jnp.int32, (1, K, K), 2))
    before = (wj > wi) | ((wj == wi) & jlt)
    shifted = jnp.sum(jnp.where(before, wj, 0.0), axis=1)
    maskf = (shifted <= 0.9).astype(jnp.float32)

    fs = lax.dot_general(q1, fn_ref[:], (((1,), (1,)), ((), ())),
                         preferred_element_type=jnp.float32)
    ef = jnp.exp(2.0 * fs)
    grp = (lax.broadcasted_iota(jnp.int32, (KM, K), 0) // M ==
           lax.broadcasted_iota(jnp.int32, (KM, K), 1)).astype(jnp.float32)
    gs = lax.dot_general(ef, grp, (((1,), (0,)), ((), ())),
                         preferred_element_type=jnp.float32)
    t1 = maskf / gs
    t2 = cw * t1
    rep1 = lax.dot_general(t1, grp, (((1,), (1,)), ((), ())),
                           preferred_element_type=jnp.float32)
    rep2 = lax.dot_general(t2, grp, (((1,), (1,)), ((), ())),
                           preferred_element_type=jnp.float32)
    fw_ref[:] = ef * rep1
    comb = ef * rep2
    prompt_ref[:] = lax.dot_general(comb, fp_ref[:], (((1,), (0,)), ((), ())),
                                    preferred_element_type=jnp.float32)
    cw_ref[:] = cw


@functools.partial(jax.jit, static_argnames=())
def kernel(cls_token, coarse_prototypes, fine_prototypes, Wq, bq, Wp, bp,
           ln_g, ln_b):
    f32 = jnp.float32
    ff = fine_prototypes.reshape(KM, D)
    cn, fn, fp = pl.pallas_call(
        _prep_kernel,
        out_shape=(
            jax.ShapeDtypeStruct((K, D), f32),
            jax.ShapeDtypeStruct((KM, D), f32),
            jax.ShapeDtypeStruct((KM, D), f32),
        ),
    )(coarse_prototypes, ff, Wp, bp.reshape(1, D))

    nblk = B // BLK
    row = lambda i: (i, 0)
    rep = lambda i: (0, 0)
    prompt, cw, fw = pl.pallas_call(
        _main_kernel,
        grid=(nblk,),
        in_specs=[
            pl.BlockSpec((BLK, D), row),
            pl.BlockSpec((D, D), rep),
            pl.BlockSpec((1, D), rep),
            pl.BlockSpec((1, D), rep),
            pl.BlockSpec((1, D), rep),
            pl.BlockSpec((K, D), rep),
            pl.BlockSpec((KM, D), rep),
            pl.BlockSpec((KM, D), rep),
        ],
        out_specs=(
            pl.BlockSpec((BLK, D), row),
            pl.BlockSpec((BLK, K), row),
            pl.BlockSpec((BLK, KM), row),
        ),
        out_shape=(
            jax.ShapeDtypeStruct((B, D), f32),
            jax.ShapeDtypeStruct((B, K), f32),
            jax.ShapeDtypeStruct((B, KM), f32),
        ),
    )(cls_token, Wq, bq.reshape(1, D), ln_g.reshape(1, D),
      ln_b.reshape(1, D), cn, fn, fp)
    return prompt, cw, fw.reshape(B, K, M)


# speedup vs baseline: 24.2399x; 1.2475x over previous
import functools

import jax
import jax.numpy as jnp
from jax import lax
from jax.experimental import pallas as pl

B = 16384
D = 128
K = 64
M = 8
KM = K * M
BLK = 512


def _prep_kernel(cp_ref, ff_ref, wp_ref, bp_ref, cn_ref, fn_ref, fp_ref):
    cp = cp_ref[:]
    cn_ref[:] = cp / jnp.maximum(
        jnp.sqrt(jnp.sum(cp * cp, axis=1, keepdims=True)), 1e-12)
    ff = ff_ref[:]
    fn_ref[:] = ff / jnp.maximum(
        jnp.sqrt(jnp.sum(ff * ff, axis=1, keepdims=True)), 1e-12)
    fp_ref[:] = lax.dot_general(
        ff, wp_ref[:], (((1,), (1,)), ((), ())),
        preferred_element_type=jnp.float32) + bp_ref[:]


def _main_kernel(cls_ref, wq_ref, bq_ref, g_ref, b_ref,
                 cn_ref, fn_ref, fp_ref,
                 prompt_ref, cw_ref, fw_ref):
    x = cls_ref[:]
    q = lax.dot_general(x, wq_ref[:], (((1,), (1,)), ((), ())),
                        preferred_element_type=jnp.float32) + bq_ref[:]
    mu = jnp.mean(q, axis=1, keepdims=True)
    c = q - mu
    var = jnp.mean(c * c, axis=1, keepdims=True)
    qn = c / jnp.sqrt(var + 1e-5) * g_ref[:] + b_ref[:]
    nrm = jnp.maximum(jnp.sqrt(jnp.sum(qn * qn, axis=1, keepdims=True)), 1e-12)
    q1 = qn / nrm

    cs = lax.dot_general(q1, cn_ref[:], (((1,), (1,)), ((), ())),
                         preferred_element_type=jnp.float32)
    ec = jnp.exp(2.0 * cs)
    cw = ec / jnp.sum(ec, axis=1, keepdims=True)

    wj = cw[:, :, None]
    wi = cw[:, None, :]
    shifted = jnp.sum(jnp.where(wj > wi, wj, 0.0), axis=1)
    maskf = (shifted <= 0.9).astype(jnp.float32)

    fs = lax.dot_general(q1, fn_ref[:], (((1,), (1,)), ((), ())),
                         preferred_element_type=jnp.float32)
    ef = jnp.exp(2.0 * fs)
    grp = (lax.broadcasted_iota(jnp.int32, (KM, K), 0) // M ==
           lax.broadcasted_iota(jnp.int32, (KM, K), 1)).astype(jnp.float32)
    gs = lax.dot_general(ef, grp, (((1,), (0,)), ((), ())),
                         preferred_element_type=jnp.float32)
    t1 = maskf / gs
    t2 = cw * t1
    rep1 = lax.dot_general(t1, grp, (((1,), (1,)), ((), ())),
                           preferred_element_type=jnp.float32)
    rep2 = lax.dot_general(t2, grp, (((1,), (1,)), ((), ())),
                           preferred_element_type=jnp.float32)
    fw_ref[:] = ef * rep1
    comb = ef * rep2
    prompt_ref[:] = lax.dot_general(comb, fp_ref[:], (((1,), (0,)), ((), ())),
                                    preferred_element_type=jnp.float32)
    cw_ref[:] = cw


@functools.partial(jax.jit, static_argnames=())
def kernel(cls_token, coarse_prototypes, fine_prototypes, Wq, bq, Wp, bp,
           ln_g, ln_b):
    f32 = jnp.float32
    ff = fine_prototypes.reshape(KM, D)
    cn, fn, fp = pl.pallas_call(
        _prep_kernel,
        out_shape=(
            jax.ShapeDtypeStruct((K, D), f32),
            jax.ShapeDtypeStruct((KM, D), f32),
            jax.ShapeDtypeStruct((KM, D), f32),
        ),
    )(coarse_prototypes, ff, Wp, bp.reshape(1, D))

    nblk = B // BLK
    row = lambda i: (i, 0)
    rep = lambda i: (0, 0)
    prompt, cw, fw = pl.pallas_call(
        _main_kernel,
        grid=(nblk,),
        in_specs=[
            pl.BlockSpec((BLK, D), row),
            pl.BlockSpec((D, D), rep),
            pl.BlockSpec((1, D), rep),
            pl.BlockSpec((1, D), rep),
            pl.BlockSpec((1, D), rep),
            pl.BlockSpec((K, D), rep),
            pl.BlockSpec((KM, D), rep),
            pl.BlockSpec((KM, D), rep),
        ],
        out_specs=(
            pl.BlockSpec((BLK, D), row),
            pl.BlockSpec((BLK, K), row),
            pl.BlockSpec((BLK, KM), row),
        ),
        out_shape=(
            jax.ShapeDtypeStruct((B, D), f32),
            jax.ShapeDtypeStruct((B, K), f32),
            jax.ShapeDtypeStruct((B, KM), f32),
        ),
    )(cls_token, Wq, bq.reshape(1, D), ln_g.reshape(1, D),
      ln_b.reshape(1, D), cn, fn, fp)
    return prompt, cw, fw.reshape(B, K, M)


# speedup vs baseline: 43.4322x; 1.7918x over previous
import functools

import jax
import jax.numpy as jnp
from jax import lax
from jax.experimental import pallas as pl

B = 16384
D = 128
K = 64
M = 8
KM = K * M
BLK = 512


def _prep_kernel(cp_ref, ff_ref, wp_ref, bp_ref, cn_ref, fn_ref, fp_ref):
    cp = cp_ref[:]
    cn_ref[:] = cp / jnp.maximum(
        jnp.sqrt(jnp.sum(cp * cp, axis=1, keepdims=True)), 1e-12)
    ff = ff_ref[:]
    fn_ref[:] = ff / jnp.maximum(
        jnp.sqrt(jnp.sum(ff * ff, axis=1, keepdims=True)), 1e-12)
    fp_ref[:] = lax.dot_general(
        ff, wp_ref[:], (((1,), (1,)), ((), ())),
        preferred_element_type=jnp.float32) + bp_ref[:]


def _main_kernel(cls_ref, wq_ref, bq_ref, g_ref, b_ref,
                 cn_ref, fn_ref, fp_ref,
                 prompt_ref, cw_ref, fw_ref):
    x = cls_ref[:]
    q = lax.dot_general(x, wq_ref[:], (((1,), (1,)), ((), ())),
                        preferred_element_type=jnp.float32) + bq_ref[:]
    mu = jnp.mean(q, axis=1, keepdims=True)
    c = q - mu
    var = jnp.mean(c * c, axis=1, keepdims=True)
    qn = c / jnp.sqrt(var + 1e-5) * g_ref[:] + b_ref[:]
    nrm = jnp.maximum(jnp.sqrt(jnp.sum(qn * qn, axis=1, keepdims=True)), 1e-12)
    q1 = qn / nrm

    cs = lax.dot_general(q1, cn_ref[:], (((1,), (1,)), ((), ())),
                         preferred_element_type=jnp.float32)
    ec = jnp.exp(2.0 * cs)
    cw = ec / jnp.sum(ec, axis=1, keepdims=True)

    cwT = cw.T
    wj = cwT[:, None, :]
    wi = cwT[None, :, :]
    shiftedT = jnp.sum(jnp.where(wj > wi, wj, 0.0), axis=0)
    maskf = (shiftedT <= 0.9).astype(jnp.float32).T

    fs = lax.dot_general(q1, fn_ref[:], (((1,), (1,)), ((), ())),
                         preferred_element_type=jnp.float32)
    ef = jnp.exp(2.0 * fs)
    grp = (lax.broadcasted_iota(jnp.int32, (KM, K), 0) // M ==
           lax.broadcasted_iota(jnp.int32, (KM, K), 1)).astype(jnp.float32)
    gs = lax.dot_general(ef, grp, (((1,), (0,)), ((), ())),
                         preferred_element_type=jnp.float32)
    t1 = maskf / gs
    t2 = cw * t1
    rep1 = lax.dot_general(t1, grp, (((1,), (1,)), ((), ())),
                           preferred_element_type=jnp.float32)
    rep2 = lax.dot_general(t2, grp, (((1,), (1,)), ((), ())),
                           preferred_element_type=jnp.float32)
    fw_ref[:] = ef * rep1
    comb = ef * rep2
    prompt_ref[:] = lax.dot_general(comb, fp_ref[:], (((1,), (0,)), ((), ())),
                                    preferred_element_type=jnp.float32)
    cw_ref[:] = cw


@functools.partial(jax.jit, static_argnames=())
def kernel(cls_token, coarse_prototypes, fine_prototypes, Wq, bq, Wp, bp,
           ln_g, ln_b):
    f32 = jnp.float32
    ff = fine_prototypes.reshape(KM, D)
    cn, fn, fp = pl.pallas_call(
        _prep_kernel,
        out_shape=(
            jax.ShapeDtypeStruct((K, D), f32),
            jax.ShapeDtypeStruct((KM, D), f32),
            jax.ShapeDtypeStruct((KM, D), f32),
        ),
    )(coarse_prototypes, ff, Wp, bp.reshape(1, D))

    nblk = B // BLK
    row = lambda i: (i, 0)
    rep = lambda i: (0, 0)
    prompt, cw, fw = pl.pallas_call(
        _main_kernel,
        grid=(nblk,),
        in_specs=[
            pl.BlockSpec((BLK, D), row),
            pl.BlockSpec((D, D), rep),
            pl.BlockSpec((1, D), rep),
            pl.BlockSpec((1, D), rep),
            pl.BlockSpec((1, D), rep),
            pl.BlockSpec((K, D), rep),
            pl.BlockSpec((KM, D), rep),
            pl.BlockSpec((KM, D), rep),
        ],
        out_specs=(
            pl.BlockSpec((BLK, D), row),
            pl.BlockSpec((BLK, K), row),
            pl.BlockSpec((BLK, KM), row),
        ),
        out_shape=(
            jax.ShapeDtypeStruct((B, D), f32),
            jax.ShapeDtypeStruct((B, K), f32),
            jax.ShapeDtypeStruct((B, KM), f32),
        ),
    )(cls_token, Wq, bq.reshape(1, D), ln_g.reshape(1, D),
      ln_b.reshape(1, D), cn, fn, fp)
    return prompt, cw, fw.reshape(B, K, M)


# speedup vs baseline: 48.2842x; 1.1117x over previous
import functools

import jax
import jax.numpy as jnp
from jax import lax
from jax.experimental import pallas as pl
from jax.experimental.pallas import tpu as pltpu

B = 16384
D = 128
K = 64
M = 8
KM = K * M
BLK = 1024


def _prep_kernel(cp_ref, ff_ref, wp_ref, bp_ref, cn_ref, fn_ref, fp_ref):
    cp = cp_ref[:]
    cn_ref[:] = cp / jnp.maximum(
        jnp.sqrt(jnp.sum(cp * cp, axis=1, keepdims=True)), 1e-12)
    ff = ff_ref[:]
    fn_ref[:] = ff / jnp.maximum(
        jnp.sqrt(jnp.sum(ff * ff, axis=1, keepdims=True)), 1e-12)
    fp_ref[:] = lax.dot_general(
        ff, wp_ref[:], (((1,), (1,)), ((), ())),
        preferred_element_type=jnp.float32) + bp_ref[:]


def _main_kernel(cls_ref, wq_ref, bq_ref, g_ref, b_ref,
                 cn_ref, fn_ref, fp_ref,
                 prompt_ref, cw_ref, fw_ref):
    x = cls_ref[:]
    q = lax.dot_general(x, wq_ref[:], (((1,), (1,)), ((), ())),
                        preferred_element_type=jnp.float32) + bq_ref[:]
    mu = jnp.mean(q, axis=1, keepdims=True)
    c = q - mu
    var = jnp.mean(c * c, axis=1, keepdims=True)
    qn = c / jnp.sqrt(var + 1e-5) * g_ref[:] + b_ref[:]
    nrm = jnp.maximum(jnp.sqrt(jnp.sum(qn * qn, axis=1, keepdims=True)), 1e-12)
    q1 = qn / nrm

    cs = lax.dot_general(q1, cn_ref[:], (((1,), (1,)), ((), ())),
                         preferred_element_type=jnp.float32)
    ec = jnp.exp(2.0 * cs)
    cw = ec / jnp.sum(ec, axis=1, keepdims=True)

    cwT = cw.T
    wj = cwT[:, None, :]
    wi = cwT[None, :, :]
    shiftedT = jnp.sum(jnp.where(wj > wi, wj, 0.0), axis=0)
    maskf = (shiftedT <= 0.9).astype(jnp.float32).T

    fs = lax.dot_general(q1, fn_ref[:], (((1,), (1,)), ((), ())),
                         preferred_element_type=jnp.float32)
    ef = jnp.exp(2.0 * fs)
    grp = (lax.broadcasted_iota(jnp.int32, (KM, K), 0) // M ==
           lax.broadcasted_iota(jnp.int32, (KM, K), 1)).astype(jnp.float32)
    gs = lax.dot_general(ef, grp, (((1,), (0,)), ((), ())),
                         preferred_element_type=jnp.float32)
    t1 = maskf / gs
    t2 = cw * t1
    rep1 = lax.dot_general(t1, grp, (((1,), (1,)), ((), ())),
                           preferred_element_type=jnp.float32)
    rep2 = lax.dot_general(t2, grp, (((1,), (1,)), ((), ())),
                           preferred_element_type=jnp.float32)
    fw_ref[:] = ef * rep1
    comb = ef * rep2
    prompt_ref[:] = lax.dot_general(comb, fp_ref[:], (((1,), (0,)), ((), ())),
                                    preferred_element_type=jnp.float32)
    cw_ref[:] = cw


@functools.partial(jax.jit, static_argnames=())
def kernel(cls_token, coarse_prototypes, fine_prototypes, Wq, bq, Wp, bp,
           ln_g, ln_b):
    f32 = jnp.float32
    ff = fine_prototypes.reshape(KM, D)
    cn, fn, fp = pl.pallas_call(
        _prep_kernel,
        out_shape=(
            jax.ShapeDtypeStruct((K, D), f32),
            jax.ShapeDtypeStruct((KM, D), f32),
            jax.ShapeDtypeStruct((KM, D), f32),
        ),
    )(coarse_prototypes, ff, Wp, bp.reshape(1, D))

    nblk = B // BLK
    row = lambda i: (i, 0)
    rep = lambda i: (0, 0)
    prompt, cw, fw = pl.pallas_call(
        _main_kernel,
        grid=(nblk,),
        in_specs=[
            pl.BlockSpec((BLK, D), row),
            pl.BlockSpec((D, D), rep),
            pl.BlockSpec((1, D), rep),
            pl.BlockSpec((1, D), rep),
            pl.BlockSpec((1, D), rep),
            pl.BlockSpec((K, D), rep),
            pl.BlockSpec((KM, D), rep),
            pl.BlockSpec((KM, D), rep),
        ],
        out_specs=(
            pl.BlockSpec((BLK, D), row),
            pl.BlockSpec((BLK, K), row),
            pl.BlockSpec((BLK, KM), row),
        ),
        out_shape=(
            jax.ShapeDtypeStruct((B, D), f32),
            jax.ShapeDtypeStruct((B, K), f32),
            jax.ShapeDtypeStruct((B, KM), f32),
        ),
        compiler_params=pltpu.CompilerParams(
            dimension_semantics=("parallel",)),
    )(cls_token, Wq, bq.reshape(1, D), ln_g.reshape(1, D),
      ln_b.reshape(1, D), cn, fn, fp)
    return prompt, cw, fw.reshape(B, K, M)


# speedup vs baseline: 75.9222x; 1.5724x over previous
import functools

import jax
import jax.numpy as jnp
from jax import lax
from jax.experimental import pallas as pl
from jax.experimental.pallas import tpu as pltpu

B = 16384
D = 128
K = 64
M = 8
KM = K * M
BLK = 1024


def _prep_kernel(cp_ref, ff_ref, wp_ref, bp_ref, cn_ref, fn_ref, fp_ref):
    cp = cp_ref[:]
    cn_ref[:] = cp / jnp.maximum(
        jnp.sqrt(jnp.sum(cp * cp, axis=1, keepdims=True)), 1e-12)
    ff = ff_ref[:]
    fn_ref[:] = ff / jnp.maximum(
        jnp.sqrt(jnp.sum(ff * ff, axis=1, keepdims=True)), 1e-12)
    fp_ref[:] = lax.dot_general(
        ff, wp_ref[:], (((1,), (1,)), ((), ())),
        preferred_element_type=jnp.float32) + bp_ref[:]


def _main_kernel(cls_ref, wq_ref, bq_ref, g_ref, b_ref,
                 cn_ref, fn_ref, fp_ref,
                 prompt_ref, cwT_ref, fwT_ref):
    x = cls_ref[:]
    q = lax.dot_general(x, wq_ref[:], (((1,), (1,)), ((), ())),
                        preferred_element_type=jnp.float32) + bq_ref[:]
    mu = jnp.mean(q, axis=1, keepdims=True)
    c = q - mu
    var = jnp.mean(c * c, axis=1, keepdims=True)
    qn = c / jnp.sqrt(var + 1e-5) * g_ref[:] + b_ref[:]
    nrm = jnp.maximum(jnp.sqrt(jnp.sum(qn * qn, axis=1, keepdims=True)), 1e-12)
    q1 = qn / nrm

    csT = lax.dot_general(cn_ref[:], q1, (((1,), (1,)), ((), ())),
                          preferred_element_type=jnp.float32)
    ecT = jnp.exp(2.0 * csT)
    cwT = ecT / jnp.sum(ecT, axis=0, keepdims=True)

    wj = cwT[:, None, :]
    wi = cwT[None, :, :]
    shiftedT = jnp.sum(jnp.where(wj > wi, wj, 0.0), axis=0)
    maskT = (shiftedT <= 0.9).astype(jnp.float32)

    fsT = lax.dot_general(fn_ref[:], q1, (((1,), (1,)), ((), ())),
                          preferred_element_type=jnp.float32)
    efT = jnp.exp(2.0 * fsT)
    grp = (lax.broadcasted_iota(jnp.int32, (KM, K), 0) // M ==
           lax.broadcasted_iota(jnp.int32, (KM, K), 1)).astype(jnp.float32)
    gsT = lax.dot_general(grp, efT, (((0,), (0,)), ((), ())),
                          preferred_element_type=jnp.float32)
    t1 = maskT / gsT
    t2 = cwT * t1
    rep1 = lax.dot_general(grp, t1, (((1,), (0,)), ((), ())),
                           preferred_element_type=jnp.float32)
    rep2 = lax.dot_general(grp, t2, (((1,), (0,)), ((), ())),
                           preferred_element_type=jnp.float32)
    fwT_ref[:] = efT * rep1
    combT = efT * rep2
    prompt_ref[:] = lax.dot_general(combT, fp_ref[:], (((0,), (0,)), ((), ())),
                                    preferred_element_type=jnp.float32)
    cwT_ref[:] = cwT


@functools.partial(jax.jit, static_argnames=())
def kernel(cls_token, coarse_prototypes, fine_prototypes, Wq, bq, Wp, bp,
           ln_g, ln_b):
    f32 = jnp.float32
    ff = fine_prototypes.reshape(KM, D)
    cn, fn, fp = pl.pallas_call(
        _prep_kernel,
        out_shape=(
            jax.ShapeDtypeStruct((K, D), f32),
            jax.ShapeDtypeStruct((KM, D), f32),
            jax.ShapeDtypeStruct((KM, D), f32),
        ),
    )(coarse_prototypes, ff, Wp, bp.reshape(1, D))

    nblk = B // BLK
    row = lambda i: (i, 0)
    col = lambda i: (0, i)
    rep = lambda i: (0, 0)
    prompt, cwT, fwT = pl.pallas_call(
        _main_kernel,
        grid=(nblk,),
        in_specs=[
            pl.BlockSpec((BLK, D), row),
            pl.BlockSpec((D, D), rep),
            pl.BlockSpec((1, D), rep),
            pl.BlockSpec((1, D), rep),
            pl.BlockSpec((1, D), rep),
            pl.BlockSpec((K, D), rep),
            pl.BlockSpec((KM, D), rep),
            pl.BlockSpec((KM, D), rep),
        ],
        out_specs=(
            pl.BlockSpec((BLK, D), row),
            pl.BlockSpec((K, BLK), col),
            pl.BlockSpec((KM, BLK), col),
        ),
        out_shape=(
            jax.ShapeDtypeStruct((B, D), f32),
            jax.ShapeDtypeStruct((K, B), f32),
            jax.ShapeDtypeStruct((KM, B), f32),
        ),
        compiler_params=pltpu.CompilerParams(
            dimension_semantics=("parallel",)),
    )(cls_token, Wq, bq.reshape(1, D), ln_g.reshape(1, D),
      ln_b.reshape(1, D), cn, fn, fp)
    return prompt, cwT.T, fwT.T.reshape(B, K, M)


# speedup vs baseline: 76.5656x; 1.0085x over previous
import functools

import jax
import jax.numpy as jnp
from jax import lax
from jax.experimental import pallas as pl
from jax.experimental.pallas import tpu as pltpu

B = 16384
D = 128
K = 64
M = 8
KM = K * M
BLK = 1024


def _prep_kernel(cp_ref, ff_ref, wp_ref, bp_ref, cn_ref, fn_ref, fp_ref):
    cp = cp_ref[:]
    cn_ref[:] = cp / jnp.maximum(
        jnp.sqrt(jnp.sum(cp * cp, axis=1, keepdims=True)), 1e-12)
    ff = ff_ref[:]
    fn_ref[:] = ff / jnp.maximum(
        jnp.sqrt(jnp.sum(ff * ff, axis=1, keepdims=True)), 1e-12)
    fp_ref[:] = lax.dot_general(
        ff, wp_ref[:], (((1,), (1,)), ((), ())),
        preferred_element_type=jnp.float32) + bp_ref[:]


def _main_kernel(cls_ref, wq_ref, bq_ref, g_ref, b_ref,
                 cn_ref, fn_ref, fp_ref,
                 prompt_ref, cwT_ref, fwT_ref):
    qT = lax.dot_general(wq_ref[:], cls_ref[:], (((1,), (1,)), ((), ())),
                         preferred_element_type=jnp.float32) + bq_ref[:]
    mu = jnp.mean(qT, axis=0, keepdims=True)
    c = qT - mu
    var = jnp.mean(c * c, axis=0, keepdims=True)
    qn = c / jnp.sqrt(var + 1e-5) * g_ref[:] + b_ref[:]
    nrm = jnp.maximum(jnp.sqrt(jnp.sum(qn * qn, axis=0, keepdims=True)), 1e-12)
    q1T = qn / nrm

    csT = lax.dot_general(cn_ref[:], q1T, (((1,), (0,)), ((), ())),
                          preferred_element_type=jnp.float32)
    ecT = jnp.exp(2.0 * csT)
    cwT = ecT / jnp.sum(ecT, axis=0, keepdims=True)

    wj = cwT[:, None, :]
    wi = cwT[None, :, :]
    shiftedT = jnp.sum(jnp.where(wj > wi, wj, 0.0), axis=0)
    maskT = (shiftedT <= 0.9).astype(jnp.float32)

    fsT = lax.dot_general(fn_ref[:], q1T, (((1,), (0,)), ((), ())),
                          preferred_element_type=jnp.float32)
    efT = jnp.exp(2.0 * fsT)
    grp = (lax.broadcasted_iota(jnp.int32, (KM, K), 0) // M ==
           lax.broadcasted_iota(jnp.int32, (KM, K), 1)).astype(jnp.float32)
    gsT = lax.dot_general(grp, efT, (((0,), (0,)), ((), ())),
                          preferred_element_type=jnp.float32)
    t1 = maskT / gsT
    t2 = cwT * t1
    rep1 = lax.dot_general(grp, t1, (((1,), (0,)), ((), ())),
                           preferred_element_type=jnp.float32)
    rep2 = lax.dot_general(grp, t2, (((1,), (0,)), ((), ())),
                           preferred_element_type=jnp.float32)
    fwT_ref[:] = efT * rep1
    combT = efT * rep2
    prompt_ref[:] = lax.dot_general(combT, fp_ref[:], (((0,), (0,)), ((), ())),
                                    preferred_element_type=jnp.float32)
    cwT_ref[:] = cwT


@functools.partial(jax.jit, static_argnames=())
def kernel(cls_token, coarse_prototypes, fine_prototypes, Wq, bq, Wp, bp,
           ln_g, ln_b):
    f32 = jnp.float32
    ff = fine_prototypes.reshape(KM, D)
    cn, fn, fp = pl.pallas_call(
        _prep_kernel,
        out_shape=(
            jax.ShapeDtypeStruct((K, D), f32),
            jax.ShapeDtypeStruct((KM, D), f32),
            jax.ShapeDtypeStruct((KM, D), f32),
        ),
    )(coarse_prototypes, ff, Wp, bp.reshape(1, D))

    nblk = B // BLK
    row = lambda i: (i, 0)
    col = lambda i: (0, i)
    rep = lambda i: (0, 0)
    prompt, cwT, fwT = pl.pallas_call(
        _main_kernel,
        grid=(nblk,),
        in_specs=[
            pl.BlockSpec((BLK, D), row),
            pl.BlockSpec((D, D), rep),
            pl.BlockSpec((D, 1), rep),
            pl.BlockSpec((D, 1), rep),
            pl.BlockSpec((D, 1), rep),
            pl.BlockSpec((K, D), rep),
            pl.BlockSpec((KM, D), rep),
            pl.BlockSpec((KM, D), rep),
        ],
        out_specs=(
            pl.BlockSpec((BLK, D), row),
            pl.BlockSpec((K, BLK), col),
            pl.BlockSpec((KM, BLK), col),
        ),
        out_shape=(
            jax.ShapeDtypeStruct((B, D), f32),
            jax.ShapeDtypeStruct((K, B), f32),
            jax.ShapeDtypeStruct((KM, B), f32),
        ),
        compiler_params=pltpu.CompilerParams(
            dimension_semantics=("parallel",)),
    )(cls_token, Wq, bq.reshape(D, 1), ln_g.reshape(D, 1),
      ln_b.reshape(D, 1), cn, fn, fp)
    return prompt, cwT.T, fwT.T.reshape(B, K, M)


# speedup vs baseline: 82.4907x; 1.0774x over previous
import functools

import jax
import jax.numpy as jnp
from jax import lax
from jax.experimental import pallas as pl
from jax.experimental.pallas import tpu as pltpu

B = 16384
D = 128
K = 64
M = 8
KM = K * M
BLK = 2048


def _prep_kernel(cp_ref, ff_ref, wp_ref, bp_ref, cn_ref, fn_ref, fp_ref):
    cp = cp_ref[:]
    cn_ref[:] = cp / jnp.maximum(
        jnp.sqrt(jnp.sum(cp * cp, axis=1, keepdims=True)), 1e-12)
    ff = ff_ref[:]
    fn_ref[:] = ff / jnp.maximum(
        jnp.sqrt(jnp.sum(ff * ff, axis=1, keepdims=True)), 1e-12)
    fp_ref[:] = lax.dot_general(
        ff, wp_ref[:], (((1,), (1,)), ((), ())),
        preferred_element_type=jnp.float32) + bp_ref[:]


def _main_kernel(cls_ref, wq_ref, bq_ref, g_ref, b_ref,
                 cn_ref, fn_ref, fp_ref,
                 prompt_ref, cwT_ref, fwT_ref):
    qT = lax.dot_general(wq_ref[:], cls_ref[:], (((1,), (1,)), ((), ())),
                         preferred_element_type=jnp.float32) + bq_ref[:]
    mu = jnp.mean(qT, axis=0, keepdims=True)
    c = qT - mu
    var = jnp.mean(c * c, axis=0, keepdims=True)
    qn = c / jnp.sqrt(var + 1e-5) * g_ref[:] + b_ref[:]
    nrm = jnp.maximum(jnp.sqrt(jnp.sum(qn * qn, axis=0, keepdims=True)), 1e-12)
    q1T = qn / nrm

    csT = lax.dot_general(cn_ref[:], q1T, (((1,), (0,)), ((), ())),
                          preferred_element_type=jnp.float32)
    ecT = jnp.exp(2.0 * csT)
    cwT = ecT / jnp.sum(ecT, axis=0, keepdims=True)

    wi = cwT[None, :, :]
    shiftedT = jnp.zeros_like(cwT)
    for jc in range(0, K, 8):
        wj = cwT[jc:jc + 8][:, None, :]
        shiftedT = shiftedT + jnp.sum(
            jnp.where(wj > wi, wj, 0.0), axis=0)
    maskT = (shiftedT <= 0.9).astype(jnp.float32)

    fsT = lax.dot_general(fn_ref[:], q1T, (((1,), (0,)), ((), ())),
                          preferred_element_type=jnp.float32)
    efT = jnp.exp(2.0 * fsT)
    grp = (lax.broadcasted_iota(jnp.int32, (KM, K), 0) // M ==
           lax.broadcasted_iota(jnp.int32, (KM, K), 1)).astype(jnp.float32)
    gsT = lax.dot_general(grp, efT, (((0,), (0,)), ((), ())),
                          preferred_element_type=jnp.float32)
    t1 = maskT / gsT
    t2 = cwT * t1
    rep1 = lax.dot_general(grp, t1, (((1,), (0,)), ((), ())),
                           preferred_element_type=jnp.float32)
    rep2 = lax.dot_general(grp, t2, (((1,), (0,)), ((), ())),
                           preferred_element_type=jnp.float32)
    fwT_ref[:] = efT * rep1
    combT = efT * rep2
    prompt_ref[:] = lax.dot_general(combT, fp_ref[:], (((0,), (0,)), ((), ())),
                                    preferred_element_type=jnp.float32)
    cwT_ref[:] = cwT


@functools.partial(jax.jit, static_argnames=())
def kernel(cls_token, coarse_prototypes, fine_prototypes, Wq, bq, Wp, bp,
           ln_g, ln_b):
    f32 = jnp.float32
    ff = fine_prototypes.reshape(KM, D)
    cn, fn, fp = pl.pallas_call(
        _prep_kernel,
        out_shape=(
            jax.ShapeDtypeStruct((K, D), f32),
            jax.ShapeDtypeStruct((KM, D), f32),
            jax.ShapeDtypeStruct((KM, D), f32),
        ),
    )(coarse_prototypes, ff, Wp, bp.reshape(1, D))

    nblk = B // BLK
    row = lambda i: (i, 0)
    col = lambda i: (0, i)
    rep = lambda i: (0, 0)
    prompt, cwT, fwT = pl.pallas_call(
        _main_kernel,
        grid=(nblk,),
        in_specs=[
            pl.BlockSpec((BLK, D), row),
            pl.BlockSpec((D, D), rep),
            pl.BlockSpec((D, 1), rep),
            pl.BlockSpec((D, 1), rep),
            pl.BlockSpec((D, 1), rep),
            pl.BlockSpec((K, D), rep),
            pl.BlockSpec((KM, D), rep),
            pl.BlockSpec((KM, D), rep),
        ],
        out_specs=(
            pl.BlockSpec((BLK, D), row),
            pl.BlockSpec((K, BLK), col),
            pl.BlockSpec((KM, BLK), col),
        ),
        out_shape=(
            jax.ShapeDtypeStruct((B, D), f32),
            jax.ShapeDtypeStruct((K, B), f32),
            jax.ShapeDtypeStruct((KM, B), f32),
        ),
        compiler_params=pltpu.CompilerParams(
            dimension_semantics=("parallel",)),
    )(cls_token, Wq, bq.reshape(D, 1), ln_g.reshape(D, 1),
      ln_b.reshape(D, 1), cn, fn, fp)
    return prompt, cwT.T, fwT.T.reshape(B, K, M)


# speedup vs baseline: 85.1247x; 1.0319x over previous
import functools

import jax
import jax.numpy as jnp
from jax import lax
from jax.experimental import pallas as pl
from jax.experimental.pallas import tpu as pltpu

B = 16384
D = 128
K = 64
M = 8
KM = K * M
BLK = 2048


def _main_kernel(cls_ref, wq_ref, bq_ref, g_ref, b_ref,
                 cp_ref, ff_ref, wp_ref, bp_ref,
                 prompt_ref, cwT_ref, fwT_ref,
                 cn_s, fn_s, fp_s):
    @pl.when(pl.program_id(0) == 0)
    def _prep():
        cp = cp_ref[:]
        cn_s[:] = cp / jnp.maximum(
            jnp.sqrt(jnp.sum(cp * cp, axis=1, keepdims=True)), 1e-12)
        ff = ff_ref[:]
        fn_s[:] = ff / jnp.maximum(
            jnp.sqrt(jnp.sum(ff * ff, axis=1, keepdims=True)), 1e-12)
        fp_s[:] = lax.dot_general(
            ff, wp_ref[:], (((1,), (1,)), ((), ())),
            preferred_element_type=jnp.float32) + bp_ref[:]

    qT = lax.dot_general(wq_ref[:], cls_ref[:], (((1,), (1,)), ((), ())),
                         preferred_element_type=jnp.float32) + bq_ref[:]
    mu = jnp.mean(qT, axis=0, keepdims=True)
    c = qT - mu
    var = jnp.mean(c * c, axis=0, keepdims=True)
    qn = c / jnp.sqrt(var + 1e-5) * g_ref[:] + b_ref[:]
    nrm = jnp.maximum(jnp.sqrt(jnp.sum(qn * qn, axis=0, keepdims=True)), 1e-12)
    q1T = qn / nrm

    csT = lax.dot_general(cn_s[:], q1T, (((1,), (0,)), ((), ())),
                          preferred_element_type=jnp.float32)
    ecT = jnp.exp(2.0 * csT)
    cwT = ecT / jnp.sum(ecT, axis=0, keepdims=True)

    wi = cwT[None, :, :]
    shiftedT = jnp.zeros_like(cwT)
    for jc in range(0, K, 8):
        wj = cwT[jc:jc + 8][:, None, :]
        shiftedT = shiftedT + jnp.sum(
            jnp.where(wj > wi, wj, 0.0), axis=0)
    maskT = (shiftedT <= 0.9).astype(jnp.float32)

    fsT = lax.dot_general(fn_s[:], q1T, (((1,), (0,)), ((), ())),
                          preferred_element_type=jnp.float32)
    efT = jnp.exp(2.0 * fsT)
    grp = (lax.broadcasted_iota(jnp.int32, (KM, K), 0) // M ==
           lax.broadcasted_iota(jnp.int32, (KM, K), 1)).astype(jnp.float32)
    gsT = lax.dot_general(grp, efT, (((0,), (0,)), ((), ())),
                          preferred_element_type=jnp.float32)
    t1 = maskT / gsT
    t2 = cwT * t1
    rep1 = lax.dot_general(grp, t1, (((1,), (0,)), ((), ())),
                           preferred_element_type=jnp.float32)
    rep2 = lax.dot_general(grp, t2, (((1,), (0,)), ((), ())),
                           preferred_element_type=jnp.float32)
    fwT_ref[:] = efT * rep1
    combT = efT * rep2
    prompt_ref[:] = lax.dot_general(combT, fp_s[:], (((0,), (0,)), ((), ())),
                                    preferred_element_type=jnp.float32)
    cwT_ref[:] = cwT


@functools.partial(jax.jit, static_argnames=())
def kernel(cls_token, coarse_prototypes, fine_prototypes, Wq, bq, Wp, bp,
           ln_g, ln_b):
    f32 = jnp.float32
    ff = fine_prototypes.reshape(KM, D)
    nblk = B // BLK
    row = lambda i: (i, 0)
    col = lambda i: (0, i)
    rep = lambda i: (0, 0)
    prompt, cwT, fwT = pl.pallas_call(
        _main_kernel,
        grid=(nblk,),
        in_specs=[
            pl.BlockSpec((BLK, D), row),
            pl.BlockSpec((D, D), rep),
            pl.BlockSpec((D, 1), rep),
            pl.BlockSpec((D, 1), rep),
            pl.BlockSpec((D, 1), rep),
            pl.BlockSpec((K, D), rep),
            pl.BlockSpec((KM, D), rep),
            pl.BlockSpec((D, D), rep),
            pl.BlockSpec((1, D), rep),
        ],
        out_specs=(
            pl.BlockSpec((BLK, D), row),
            pl.BlockSpec((K, BLK), col),
            pl.BlockSpec((KM, BLK), col),
        ),
        out_shape=(
            jax.ShapeDtypeStruct((B, D), f32),
            jax.ShapeDtypeStruct((K, B), f32),
            jax.ShapeDtypeStruct((KM, B), f32),
        ),
        scratch_shapes=[
            pltpu.VMEM((K, D), f32),
            pltpu.VMEM((KM, D), f32),
            pltpu.VMEM((KM, D), f32),
        ],
        compiler_params=pltpu.CompilerParams(
            dimension_semantics=("arbitrary",)),
    )(cls_token, Wq, bq.reshape(D, 1), ln_g.reshape(D, 1),
      ln_b.reshape(D, 1), coarse_prototypes, ff, Wp, bp.reshape(1, D))
    return prompt, cwT.T, fwT.T.reshape(B, K, M)
